# Initial kernel scaffold; baseline (speedup 1.0000x reference)
import jax
import jax.numpy as jnp
from jax.experimental import pallas as pl


def kernel(x, edge_index, batch, edge_weights, W1, b1, W2, b2, W3, b3, Wl, bl):
    raise NotImplementedError("write your pallas kernel here")


# speedup vs baseline: 11.2478x; 11.2478x over previous
import dataclasses
import functools

import jax
import jax.numpy as jnp
from jax.experimental import pallas as pl
from jax.experimental.pallas import tpu as pltpu
from jax.experimental.pallas import tpu_sc as plsc

N = 10000
E = 320000
F_IN = 128
H = 20
B = 64
C = 10

NW = 32
EPW = E // NW
CH = 8000
LANES = 16

_MESH = plsc.VectorSubcoreMesh(core_axis_name="c", subcore_axis_name="s")

_SC_CP = pltpu.CompilerParams()
if "needs_layout_passes" in pltpu.CompilerParams.__dataclass_fields__:
    _SC_CP = dataclasses.replace(_SC_CP, needs_layout_passes=False)


def _wid():
    return jax.lax.axis_index("s") * 2 + jax.lax.axis_index("c")


@functools.partial(
    pl.kernel,
    out_type=jax.ShapeDtypeStruct((NW, N), jnp.float32),
    mesh=_MESH,
    scratch_types=[
        pltpu.VMEM((N,), jnp.float32),
        pltpu.VMEM((EPW,), jnp.int32),
        pltpu.VMEM((EPW,), jnp.float32),
    ],
    compiler_params=_SC_CP,
)
def _sc_deg(dst_hbm, ew_hbm, out_hbm, acc, dv, wv):
    wid = _wid()
    base = wid * EPW
    pltpu.sync_copy(dst_hbm.at[pl.ds(base, EPW)], dv)
    pltpu.sync_copy(ew_hbm.at[pl.ds(base, EPW)], wv)

    @pl.loop(0, N, step=LANES)
    def _(i):
        acc[pl.ds(i, LANES)] = jnp.zeros((LANES,), jnp.float32)

    @pl.loop(0, EPW, step=LANES)
    def _(i):
        d = dv[pl.ds(i, LANES)]
        w = wv[pl.ds(i, LANES)]
        plsc.addupdate_scatter(acc, [d], w)

    pltpu.sync_copy(acc, out_hbm.at[wid])


@functools.partial(
    pl.kernel,
    out_type=jax.ShapeDtypeStruct((E,), jnp.float32),
    mesh=_MESH,
    scratch_types=[
        pltpu.VMEM((N,), jnp.float32),
        pltpu.VMEM((EPW,), jnp.int32),
        pltpu.VMEM((EPW,), jnp.int32),
        pltpu.VMEM((EPW,), jnp.float32),
        pltpu.VMEM((EPW,), jnp.float32),
    ],
    compiler_params=_SC_CP,
)
def _sc_norm(src_hbm, dst_hbm, ew_hbm, dinv_hbm, out_hbm, dn, sv, dv, wv, ov):
    wid = _wid()
    base = wid * EPW
    pltpu.sync_copy(dinv_hbm.at[0], dn)
    pltpu.sync_copy(src_hbm.at[pl.ds(base, EPW)], sv)
    pltpu.sync_copy(dst_hbm.at[pl.ds(base, EPW)], dv)
    pltpu.sync_copy(ew_hbm.at[pl.ds(base, EPW)], wv)

    @pl.loop(0, EPW, step=LANES)
    def _(i):
        s = sv[pl.ds(i, LANES)]
        d = dv[pl.ds(i, LANES)]
        w = wv[pl.ds(i, LANES)]
        a = plsc.load_gather(dn, [s])
        b = plsc.load_gather(dn, [d])
        ov[pl.ds(i, LANES)] = a * w * b

    pltpu.sync_copy(ov, out_hbm.at[pl.ds(base, EPW)])


@functools.partial(
    pl.kernel,
    out_type=jax.ShapeDtypeStruct((H, N), jnp.float32),
    mesh=_MESH,
    scratch_types=[
        pltpu.VMEM((N,), jnp.float32),
        pltpu.VMEM((N,), jnp.float32),
        pltpu.VMEM((CH,), jnp.int32),
        pltpu.VMEM((CH,), jnp.int32),
        pltpu.VMEM((CH,), jnp.float32),
    ],
    compiler_params=_SC_CP,
)
def _sc_msg(ht_hbm, src_hbm, dst_hbm, norm_hbm, out_hbm,
            hcol, acc, sv, dv, nv):
    wid = _wid()

    @pl.when(wid < H)
    def _():
        pltpu.sync_copy(ht_hbm.at[wid], hcol)

        @pl.loop(0, N, step=LANES)
        def _(i):
            acc[pl.ds(i, LANES)] = jnp.zeros((LANES,), jnp.float32)

        @pl.loop(0, E, step=CH)
        def _(c):
            pltpu.sync_copy(src_hbm.at[pl.ds(c, CH)], sv)
            pltpu.sync_copy(dst_hbm.at[pl.ds(c, CH)], dv)
            pltpu.sync_copy(norm_hbm.at[pl.ds(c, CH)], nv)

            @pl.loop(0, CH, step=LANES)
            def _(i):
                s = sv[pl.ds(i, LANES)]
                d = dv[pl.ds(i, LANES)]
                nm = nv[pl.ds(i, LANES)]
                vals = plsc.load_gather(hcol, [s]) * nm
                plsc.addupdate_scatter(acc, [d], vals)

        pltpu.sync_copy(acc, out_hbm.at[wid])


def _tc_prep(deg_parts, x, W1):
    def body(parts_ref, x_ref, w_ref, dinv_ref, ht_ref):
        deg = jnp.sum(parts_ref[...], axis=0, keepdims=True) + 1.0
        dinv_ref[...] = jnp.where(deg > 0, 1.0 / jnp.sqrt(deg), 0.0)
        ht_ref[...] = jax.lax.dot_general(
            w_ref[...], x_ref[...], (((0,), (1,)), ((), ())),
            preferred_element_type=jnp.float32)

    return pl.pallas_call(
        body,
        out_shape=(
            jax.ShapeDtypeStruct((1, N), jnp.float32),
            jax.ShapeDtypeStruct((H, N), jnp.float32),
        ),
    )(deg_parts, x, W1)


def _epilogue(msg, ht, dinv, b_col):
    tmp = msg + dinv * dinv * ht + b_col
    ss = jnp.sum(tmp * tmp, axis=0, keepdims=True)
    nrm = jnp.maximum(jnp.sqrt(ss), 1e-12)
    return jnp.maximum(tmp / nrm, 0.0)


def _tc_mid(msg, ht, dinv, b_col, Wn):
    def body(m_ref, h_ref, di_ref, b_ref, w_ref, o_ref):
        emb = _epilogue(m_ref[...], h_ref[...], di_ref[...], b_ref[...])
        o_ref[...] = jax.lax.dot_general(
            w_ref[...], emb, (((0,), (0,)), ((), ())),
            preferred_element_type=jnp.float32)

    return pl.pallas_call(
        body,
        out_shape=jax.ShapeDtypeStruct((H, N), jnp.float32),
    )(msg, ht, dinv, b_col, Wn)


def _tc_final(msg, ht, dinv, b_col, batch2d, Wl, bl):
    def body(m_ref, h_ref, di_ref, b_ref, bt_ref, wl_ref, bl_ref, o_ref,
             feats_ref):
        emb = _epilogue(m_ref[...], h_ref[...], di_ref[...], b_ref[...])
        bt = bt_ref[...]
        gids = jax.lax.broadcasted_iota(jnp.int32, (B, 1), 0)
        onehot = (bt == gids).astype(jnp.float32)
        cnt = jnp.sum(onehot, axis=1)
        gsum_t = jax.lax.dot_general(
            emb, onehot, (((1,), (1,)), ((), ())),
            preferred_element_type=jnp.float32)
        gmean_t = gsum_t / jnp.maximum(cnt, 1.0)[None, :]
        for g in range(B):
            mg = jnp.where(bt == g, emb, 0.0)
            feats_ref[g, pl.ds(0, H)] = jnp.max(mg, axis=1)
            feats_ref[g, pl.ds(H, H)] = gmean_t[:, g]
        o_ref[...] = jnp.dot(feats_ref[...], wl_ref[...],
                             preferred_element_type=jnp.float32) + bl_ref[...]

    return pl.pallas_call(
        body,
        out_shape=jax.ShapeDtypeStruct((B, C), jnp.float32),
        scratch_shapes=[pltpu.VMEM((B, 2 * H), jnp.float32)],
    )(msg, ht, dinv, b_col, batch2d, Wl, bl)


def kernel(x, edge_index, batch, edge_weights, W1, b1, W2, b2, W3, b3, Wl, bl):
    src = edge_index[0]
    dst = edge_index[1]
    batch2d = batch.reshape(1, N)

    deg_parts = _sc_deg(dst, edge_weights)
    dinv, ht1 = _tc_prep(deg_parts, x, W1)
    norm = _sc_norm(src, dst, edge_weights, dinv)

    msg1 = _sc_msg(ht1, src, dst, norm)
    ht2 = _tc_mid(msg1, ht1, dinv, b1.reshape(H, 1), W2)
    msg2 = _sc_msg(ht2, src, dst, norm)
    ht3 = _tc_mid(msg2, ht2, dinv, b2.reshape(H, 1), W3)
    msg3 = _sc_msg(ht3, src, dst, norm)
    return _tc_final(msg3, ht3, dinv, b3.reshape(H, 1), batch2d, Wl, bl)


# speedup vs baseline: 37.3564x; 3.3212x over previous
import dataclasses
import functools

import jax
import jax.numpy as jnp
from jax.experimental import pallas as pl
from jax.experimental.pallas import tpu as pltpu
from jax.experimental.pallas import tpu_sc as plsc

N = 10000
E = 320000
F_IN = 128
H = 20
B = 64
C = 10

NW = 32
EPW = E // NW
CH = 8000
LANES = 16

_MESH = plsc.VectorSubcoreMesh(core_axis_name="c", subcore_axis_name="s")

_SC_CP = pltpu.CompilerParams()
if "needs_layout_passes" in pltpu.CompilerParams.__dataclass_fields__:
    _SC_CP = dataclasses.replace(_SC_CP, needs_layout_passes=False)


def _wid():
    return jax.lax.axis_index("s") * 2 + jax.lax.axis_index("c")


@functools.partial(
    pl.kernel,
    out_type=jax.ShapeDtypeStruct((NW, N), jnp.float32),
    mesh=_MESH,
    scratch_types=[
        pltpu.VMEM((N,), jnp.float32),
        pltpu.VMEM((EPW,), jnp.int32),
        pltpu.VMEM((EPW,), jnp.float32),
    ],
    compiler_params=_SC_CP,
)
def _sc_deg(dst_hbm, ew_hbm, out_hbm, acc, dv, wv):
    wid = _wid()
    base = wid * EPW
    pltpu.sync_copy(dst_hbm.at[pl.ds(base, EPW)], dv)
    pltpu.sync_copy(ew_hbm.at[pl.ds(base, EPW)], wv)

    @plsc.parallel_loop(0, N, LANES, unroll=8)
    def _(i):
        acc[pl.ds(i, LANES)] = jnp.zeros((LANES,), jnp.float32)

    @plsc.parallel_loop(0, EPW, LANES, unroll=8)
    def _(i):
        d = dv[pl.ds(i, LANES)]
        w = wv[pl.ds(i, LANES)]
        plsc.addupdate_scatter(acc, [d], w)

    pltpu.sync_copy(acc, out_hbm.at[wid])


@functools.partial(
    pl.kernel,
    out_type=jax.ShapeDtypeStruct((E,), jnp.float32),
    mesh=_MESH,
    scratch_types=[
        pltpu.VMEM((N,), jnp.float32),
        pltpu.VMEM((EPW,), jnp.int32),
        pltpu.VMEM((EPW,), jnp.int32),
        pltpu.VMEM((EPW,), jnp.float32),
        pltpu.VMEM((EPW,), jnp.float32),
    ],
    compiler_params=_SC_CP,
)
def _sc_norm(src_hbm, dst_hbm, ew_hbm, dinv_hbm, out_hbm, dn, sv, dv, wv, ov):
    wid = _wid()
    base = wid * EPW
    pltpu.sync_copy(dinv_hbm.at[0], dn)
    pltpu.sync_copy(src_hbm.at[pl.ds(base, EPW)], sv)
    pltpu.sync_copy(dst_hbm.at[pl.ds(base, EPW)], dv)
    pltpu.sync_copy(ew_hbm.at[pl.ds(base, EPW)], wv)

    @plsc.parallel_loop(0, EPW, LANES, unroll=8)
    def _(i):
        s = sv[pl.ds(i, LANES)]
        d = dv[pl.ds(i, LANES)]
        w = wv[pl.ds(i, LANES)]
        a = plsc.load_gather(dn, [s])
        b = plsc.load_gather(dn, [d])
        ov[pl.ds(i, LANES)] = a * w * b

    pltpu.sync_copy(ov, out_hbm.at[pl.ds(base, EPW)])


@functools.partial(
    pl.kernel,
    out_type=jax.ShapeDtypeStruct((H, N), jnp.float32),
    mesh=_MESH,
    scratch_types=[
        pltpu.VMEM((N,), jnp.float32),
        pltpu.VMEM((N,), jnp.float32),
        pltpu.VMEM((CH,), jnp.int32),
        pltpu.VMEM((CH,), jnp.int32),
        pltpu.VMEM((CH,), jnp.float32),
        pltpu.VMEM((CH,), jnp.int32),
        pltpu.VMEM((CH,), jnp.int32),
        pltpu.VMEM((CH,), jnp.float32),
        pltpu.SemaphoreType.DMA,
        pltpu.SemaphoreType.DMA,
    ],
    compiler_params=_SC_CP,
)
def _sc_msg(ht_hbm, src_hbm, dst_hbm, norm_hbm, out_hbm,
            hcol, acc, sv0, dv0, nv0, sv1, dv1, nv1, sem0, sem1):
    wid = _wid()

    def start(c, sv, dv, nv, sem):
        pltpu.async_copy(src_hbm.at[pl.ds(c, CH)], sv, sem)
        pltpu.async_copy(dst_hbm.at[pl.ds(c, CH)], dv, sem)
        pltpu.async_copy(norm_hbm.at[pl.ds(c, CH)], nv, sem)

    def drain(sv, dv, nv, sem):
        pltpu.make_async_copy(src_hbm.at[pl.ds(0, CH)], sv, sem).wait()
        pltpu.make_async_copy(dst_hbm.at[pl.ds(0, CH)], dv, sem).wait()
        pltpu.make_async_copy(norm_hbm.at[pl.ds(0, CH)], nv, sem).wait()

    def process(sv, dv, nv):
        @plsc.parallel_loop(0, CH, LANES, unroll=8)
        def _(i):
            s = sv[pl.ds(i, LANES)]
            d = dv[pl.ds(i, LANES)]
            nm = nv[pl.ds(i, LANES)]
            vals = plsc.load_gather(hcol, [s]) * nm
            plsc.addupdate_scatter(acc, [d], vals)

    @pl.when(wid < H)
    def _():
        start(0, sv0, dv0, nv0, sem0)
        pltpu.sync_copy(ht_hbm.at[wid], hcol)

        @plsc.parallel_loop(0, N, LANES, unroll=8)
        def _(i):
            acc[pl.ds(i, LANES)] = jnp.zeros((LANES,), jnp.float32)

        @pl.loop(0, E, step=2 * CH)
        def _(c):
            start(c + CH, sv1, dv1, nv1, sem1)
            drain(sv0, dv0, nv0, sem0)
            process(sv0, dv0, nv0)

            @pl.when(c + 2 * CH < E)
            def _():
                start(c + 2 * CH, sv0, dv0, nv0, sem0)

            drain(sv1, dv1, nv1, sem1)
            process(sv1, dv1, nv1)

        pltpu.sync_copy(acc, out_hbm.at[wid])


def _tc_prep(deg_parts, x, W1):
    def body(parts_ref, x_ref, w_ref, dinv_ref, ht_ref):
        deg = jnp.sum(parts_ref[...], axis=0, keepdims=True) + 1.0
        dinv_ref[...] = jnp.where(deg > 0, 1.0 / jnp.sqrt(deg), 0.0)
        ht_ref[...] = jax.lax.dot_general(
            w_ref[...], x_ref[...], (((0,), (1,)), ((), ())),
            preferred_element_type=jnp.float32)

    return pl.pallas_call(
        body,
        out_shape=(
            jax.ShapeDtypeStruct((1, N), jnp.float32),
            jax.ShapeDtypeStruct((H, N), jnp.float32),
        ),
    )(deg_parts, x, W1)


def _epilogue(msg, ht, dinv, b_col):
    tmp = msg + dinv * dinv * ht + b_col
    ss = jnp.sum(tmp * tmp, axis=0, keepdims=True)
    nrm = jnp.maximum(jnp.sqrt(ss), 1e-12)
    return jnp.maximum(tmp / nrm, 0.0)


def _tc_mid(msg, ht, dinv, b_col, Wn):
    def body(m_ref, h_ref, di_ref, b_ref, w_ref, o_ref):
        emb = _epilogue(m_ref[...], h_ref[...], di_ref[...], b_ref[...])
        o_ref[...] = jax.lax.dot_general(
            w_ref[...], emb, (((0,), (0,)), ((), ())),
            preferred_element_type=jnp.float32)

    return pl.pallas_call(
        body,
        out_shape=jax.ShapeDtypeStruct((H, N), jnp.float32),
    )(msg, ht, dinv, b_col, Wn)


def _tc_final(msg, ht, dinv, b_col, batch2d, Wl, bl):
    def body(m_ref, h_ref, di_ref, b_ref, bt_ref, wl_ref, bl_ref, o_ref,
             feats_ref):
        emb = _epilogue(m_ref[...], h_ref[...], di_ref[...], b_ref[...])
        bt = bt_ref[...]
        gids = jax.lax.broadcasted_iota(jnp.int32, (B, 1), 0)
        onehot = (bt == gids).astype(jnp.float32)
        cnt = jnp.sum(onehot, axis=1)
        gsum_t = jax.lax.dot_general(
            emb, onehot, (((1,), (1,)), ((), ())),
            preferred_element_type=jnp.float32)
        gmean_t = gsum_t / jnp.maximum(cnt, 1.0)[None, :]
        for g in range(B):
            mg = jnp.where(bt == g, emb, 0.0)
            feats_ref[g, pl.ds(0, H)] = jnp.max(mg, axis=1)
            feats_ref[g, pl.ds(H, H)] = gmean_t[:, g]
        o_ref[...] = jnp.dot(feats_ref[...], wl_ref[...],
                             preferred_element_type=jnp.float32) + bl_ref[...]

    return pl.pallas_call(
        body,
        out_shape=jax.ShapeDtypeStruct((B, C), jnp.float32),
        scratch_shapes=[pltpu.VMEM((B, 2 * H), jnp.float32)],
    )(msg, ht, dinv, b_col, batch2d, Wl, bl)


def kernel(x, edge_index, batch, edge_weights, W1, b1, W2, b2, W3, b3, Wl, bl):
    src = edge_index[0]
    dst = edge_index[1]
    batch2d = batch.reshape(1, N)

    deg_parts = _sc_deg(dst, edge_weights)
    dinv, ht1 = _tc_prep(deg_parts, x, W1)
    norm = _sc_norm(src, dst, edge_weights, dinv)

    msg1 = _sc_msg(ht1, src, dst, norm)
    ht2 = _tc_mid(msg1, ht1, dinv, b1.reshape(H, 1), W2)
    msg2 = _sc_msg(ht2, src, dst, norm)
    ht3 = _tc_mid(msg2, ht2, dinv, b2.reshape(H, 1), W3)
    msg3 = _sc_msg(ht3, src, dst, norm)
    return _tc_final(msg3, ht3, dinv, b3.reshape(H, 1), batch2d, Wl, bl)
